# baseline (speedup 1.0000x reference)
import functools

import jax
import jax.numpy as jnp
from jax import lax
from jax.experimental import pallas as pl
from jax.experimental.pallas import tpu as pltpu
from jax.experimental.pallas import tpu_sc as plsc

BLK = 256
NC, NS = 2, 16
NW = NC * NS
CH = 32


def _router_meta_body(x_ref, wr_ref, pos_ref, w01_ref, be_ref, aux_ref,
                      xb_ref, *, n_tokens, n_experts, nb_tot):
    x = x_ref[...]
    xb_ref[...] = x.astype(jnp.bfloat16).reshape(xb_ref.shape)

    logits = lax.dot_general(
        x, wr_ref[...], (((1,), (1,)), ((), ())),
        preferred_element_type=jnp.float32)

    e_iota = lax.broadcasted_iota(jnp.int32, logits.shape, 1)
    m1 = jnp.max(logits, axis=-1, keepdims=True)
    i1 = jnp.min(jnp.where(logits == m1, e_iota, n_experts), axis=-1,
                 keepdims=True)
    oh1 = (e_iota == i1).astype(jnp.float32)
    masked = jnp.where(e_iota == i1, -jnp.inf, logits)
    m2 = jnp.max(masked, axis=-1, keepdims=True)
    i2 = jnp.min(jnp.where(masked == m2, e_iota, n_experts), axis=-1,
                 keepdims=True)
    oh2 = (e_iota == i2).astype(jnp.float32)
    w2 = 1.0 / (1.0 + jnp.exp(m1 - m2))
    w1 = 1.0 - w2
    w01_ref[...] = jnp.concatenate([w1, w2], axis=1)

    cnt = oh1 + oh2
    c = cnt
    s = 1
    while s < n_tokens:
        c = c + jnp.concatenate(
            [jnp.zeros((s, n_experts), jnp.float32), c[:-s, :]], axis=0)
        s *= 2
    c_excl = c - cnt
    counts = c[n_tokens - 1:n_tokens, :]
    ci = counts.astype(jnp.int32)
    ca = ((ci + (BLK - 1)) // BLK) * BLK
    off = ca
    s = 1
    while s < n_experts:
        off = off + jnp.concatenate(
            [jnp.zeros((1, s), jnp.int32), off[:, :-s]], axis=1)
        s *= 2
    off_excl_f = (off - ca).astype(jnp.float32)

    slot = off_excl_f + c_excl
    p0 = jnp.sum(oh1 * slot, axis=1, keepdims=True)
    p1 = jnp.sum(oh2 * slot, axis=1, keepdims=True)
    pos_ref[...] = jnp.concatenate([p0, p1], axis=1).astype(jnp.int32)

    bstart = lax.broadcasted_iota(jnp.int32, (nb_tot, n_experts), 0) * BLK
    be_ref[...] = jnp.sum(
        (jnp.broadcast_to(off, (nb_tot, n_experts)) <= bstart
         ).astype(jnp.int32), axis=1, keepdims=True)

    z = jnp.exp(logits - m1)
    sc = z / jnp.sum(z, axis=-1, keepdims=True)
    pi_sum = jnp.sum(sc, axis=0, keepdims=True)
    aux_ref[...] = (jnp.sum(pi_sum * counts)
                    / float(n_tokens * n_tokens)).reshape(1, 1)


def _gmm_body(be_ref, xp_ref, w_ref, y_ref, wb_ref, *, n_experts):
    i = pl.program_id(0)
    be = be_ref[i]
    prev = be_ref[jnp.maximum(i - 1, 0)]

    @pl.when((be < n_experts) & ((i == 0) | (be != prev)))
    def _cast():
        wb_ref[...] = w_ref[0].astype(jnp.bfloat16)

    @pl.when(be < n_experts)
    def _():
        blk = xp_ref.shape[0]
        xb = xp_ref[...].reshape(blk, xp_ref.shape[1] * xp_ref.shape[2])
        acc = lax.dot_general(xb, wb_ref[...], (((1,), (1,)), ((), ())),
                              preferred_element_type=jnp.float32)
        y_ref[...] = acc.astype(jnp.bfloat16).reshape(y_ref.shape)


def _sharedfinal_body(xb_ref, ws_ref, y0_ref, y1_ref, w01_ref, out_ref,
                      wsb_ref):
    @pl.when(pl.program_id(0) == 0)
    def _cast():
        wsb_ref[...] = (ws_ref[0] + ws_ref[1]).astype(jnp.bfloat16)

    blk = out_ref.shape[0]
    d = out_ref.shape[1]
    xb = xb_ref[...].reshape(blk, d)
    acc = lax.dot_general(xb, wsb_ref[...], (((1,), (1,)), ((), ())),
                          preferred_element_type=jnp.float32)
    w01 = w01_ref[...]
    y0 = y0_ref[...].reshape(blk, d).astype(jnp.float32)
    y1 = y1_ref[...].reshape(blk, d).astype(jnp.float32)
    out_ref[...] = acc + w01[:, 0:1] * y0 + w01[:, 1:2] * y1


def _make_dispatch(n_tokens, d, nk_pad):
    tpw = n_tokens // NW
    nch = tpw // CH
    dsub = d // 256
    mesh = plsc.VectorSubcoreMesh(core_axis_name="c", subcore_axis_name="s")

    @functools.partial(
        pl.kernel, mesh=mesh,
        out_type=jax.ShapeDtypeStruct((nk_pad, dsub, 128), jnp.int32),
        scratch_types=[
            pltpu.VMEM((nch, 2, CH), jnp.int32),
            pltpu.VMEM((CH, dsub, 128), jnp.int32),
            pltpu.VMEM((CH, dsub, 128), jnp.int32),
        ] + [pltpu.SemaphoreType.DMA] * 6,
    )
    def dispatch(x_hbm, pos_hbm, xp_hbm, idx_v, buf0, buf1,
                 sr0, sr1, sw00, sw01, sw10, sw11):
        wid = lax.axis_index("s") * NC + lax.axis_index("c")
        base = wid * tpw
        pltpu.sync_copy(pos_hbm.at[wid], idx_v)
        bufs = (buf0, buf1)
        sr = (sr0, sr1)
        sw = ((sw00, sw01), (sw10, sw11))
        reads = [None] * nch
        writes = [None] * nch
        reads[0] = pltpu.async_copy(x_hbm.at[pl.ds(base, CH)], bufs[0], sr[0])
        for c in range(nch):
            b = c % 2
            reads[c].wait()
            if c + 1 < nch:
                if c >= 1:
                    writes[c - 1][0].wait()
                    writes[c - 1][1].wait()
                reads[c + 1] = pltpu.async_copy(
                    x_hbm.at[pl.ds(base + (c + 1) * CH, CH)],
                    bufs[1 - b], sr[1 - b])
            writes[c] = (
                pltpu.async_copy(bufs[b], xp_hbm.at[idx_v.at[c, 0]], sw[b][0]),
                pltpu.async_copy(bufs[b], xp_hbm.at[idx_v.at[c, 1]], sw[b][1]))
        writes[nch - 1][0].wait()
        writes[nch - 1][1].wait()
        if nch >= 2:
            writes[nch - 2][0].wait()
            writes[nch - 2][1].wait()

    return dispatch


def _make_combine(n_tokens, d, nk_pad):
    tpw = n_tokens // NW
    nch = tpw // CH
    dsub = d // 256
    mesh = plsc.VectorSubcoreMesh(core_axis_name="c", subcore_axis_name="s")

    @functools.partial(
        pl.kernel, mesh=mesh,
        out_type=(jax.ShapeDtypeStruct((n_tokens, dsub, 128), jnp.int32),
                  jax.ShapeDtypeStruct((n_tokens, dsub, 128), jnp.int32)),
        scratch_types=[
            pltpu.VMEM((nch, 2 * CH), jnp.int32),
            pltpu.VMEM((2 * CH, dsub, 128), jnp.int32),
            pltpu.VMEM((2 * CH, dsub, 128), jnp.int32),
        ] + [pltpu.SemaphoreType.DMA] * 6,
    )
    def combine(y_hbm, pos_hbm, y0_hbm, y1_hbm, idx_v, gbuf0, gbuf1,
                sg0, sg1, sw00, sw01, sw10, sw11):
        wid = lax.axis_index("s") * NC + lax.axis_index("c")
        base = wid * tpw
        pltpu.sync_copy(pos_hbm.at[wid], idx_v)
        gbufs = (gbuf0, gbuf1)
        sg = (sg0, sg1)
        sw = ((sw00, sw01), (sw10, sw11))
        reads = [None] * nch
        writes = [None] * nch
        reads[0] = pltpu.async_copy(y_hbm.at[idx_v.at[0]], gbufs[0], sg[0])
        for c in range(nch):
            b = c % 2
            reads[c].wait()
            if c + 1 < nch:
                if c >= 1:
                    writes[c - 1][0].wait()
                    writes[c - 1][1].wait()
                reads[c + 1] = pltpu.async_copy(
                    y_hbm.at[idx_v.at[c + 1]], gbufs[1 - b], sg[1 - b])
            dst = pl.ds(base + c * CH, CH)
            writes[c] = (
                pltpu.async_copy(gbufs[b].at[pl.ds(0, CH)],
                                 y0_hbm.at[dst], sw[b][0]),
                pltpu.async_copy(gbufs[b].at[pl.ds(CH, CH)],
                                 y1_hbm.at[dst], sw[b][1]))
        writes[nch - 1][0].wait()
        writes[nch - 1][1].wait()
        if nch >= 2:
            writes[nch - 2][0].wait()
            writes[nch - 2][1].wait()

    return combine


def _router_meta(x, W_router, nb_tot):
    n_tokens, d = x.shape
    n_experts = W_router.shape[0]
    return pl.pallas_call(
        functools.partial(_router_meta_body, n_tokens=n_tokens,
                          n_experts=n_experts, nb_tot=nb_tot),
        out_shape=[
            jax.ShapeDtypeStruct((n_tokens, 2), jnp.int32),
            jax.ShapeDtypeStruct((n_tokens, 2), jnp.float32),
            jax.ShapeDtypeStruct((nb_tot, 1), jnp.int32),
            jax.ShapeDtypeStruct((1, 1), jnp.float32),
            jax.ShapeDtypeStruct((n_tokens, d // 128, 128), jnp.bfloat16),
        ],
    )(x, W_router)


def _gmm(be, x_perm, we, n_experts, d):
    nb_tot = be.shape[0]
    dsub = d // 128
    grid_spec = pltpu.PrefetchScalarGridSpec(
        num_scalar_prefetch=1,
        grid=(nb_tot,),
        in_specs=[
            pl.BlockSpec((BLK, dsub, 128), lambda i, be_r: (i, 0, 0)),
            pl.BlockSpec((1, d, d),
                         lambda i, be_r: (jnp.minimum(be_r[i], n_experts - 1),
                                          0, 0)),
        ],
        out_specs=pl.BlockSpec((BLK, dsub, 128), lambda i, be_r: (i, 0, 0)),
        scratch_shapes=[pltpu.VMEM((d, d), jnp.bfloat16)],
    )
    return pl.pallas_call(
        functools.partial(_gmm_body, n_experts=n_experts),
        grid_spec=grid_spec,
        out_shape=jax.ShapeDtypeStruct((x_perm.shape[0], dsub, 128),
                                       jnp.bfloat16),
    )(be, x_perm, we)


def _sharedfinal(xb, ws, y0, y1, w01):
    n_tokens = xb.shape[0]
    d = xb.shape[1] * xb.shape[2]
    dsub = d // 128
    blk = 512
    return pl.pallas_call(
        _sharedfinal_body,
        grid=(n_tokens // blk,),
        in_specs=[
            pl.BlockSpec((blk, dsub, 128), lambda i: (i, 0, 0)),
            pl.BlockSpec((2, d, d), lambda i: (0, 0, 0)),
            pl.BlockSpec((blk, dsub, 128), lambda i: (i, 0, 0)),
            pl.BlockSpec((blk, dsub, 128), lambda i: (i, 0, 0)),
            pl.BlockSpec((blk, 2), lambda i: (i, 0)),
        ],
        out_specs=pl.BlockSpec((blk, d), lambda i: (i, 0)),
        out_shape=jax.ShapeDtypeStruct((n_tokens, d), jnp.float32),
        scratch_shapes=[pltpu.VMEM((d, d), jnp.bfloat16)],
    )(xb, ws, y0, y1, w01)


def _pack_i32(a):
    n = a.shape[0]
    d = a.shape[1] * a.shape[2]
    return lax.bitcast_convert_type(
        a.reshape(n, d // 2, 2), jnp.int32).reshape(n, d // 256, 128)


def _unpack_bf16(a):
    n = a.shape[0]
    d = 2 * a.shape[1] * a.shape[2]
    return lax.bitcast_convert_type(
        a.reshape(n, d // 2), jnp.bfloat16).reshape(n, d // 128, 128)


def kernel(feat, W_router, W_shared, W_experts):
    B, S, d = feat.shape
    N = B * S
    E = W_router.shape[0]
    topk = 2
    nb_tot = (N * topk) // BLK + E
    nk_pad = nb_tot * BLK

    x = feat.reshape(N, d)
    ws = W_shared.reshape(-1, d, d)

    pos, w01, be2d, aux, xb = _router_meta(x, W_router, nb_tot)
    tpw = N // NW
    nch = tpw // CH
    pos_sc = pos.reshape(NW, nch, CH, 2).transpose(0, 1, 3, 2)
    pos_disp = pos_sc
    pos_comb = pos_sc.reshape(NW, nch, 2 * CH)
    be = be2d.reshape(nb_tot)

    x_perm = _unpack_bf16(_make_dispatch(N, d, nk_pad)(_pack_i32(xb),
                                                       pos_disp))
    y = _gmm(be, x_perm, W_experts, E, d)
    y0, y1 = _make_combine(N, d, nk_pad)(_pack_i32(y), pos_comb)
    out = _sharedfinal(xb, ws, _unpack_bf16(y0), _unpack_bf16(y1), w01)
    return out.reshape(B, S, d), aux[0, 0]


# speedup vs baseline: 5.6653x; 5.6653x over previous
import functools

import jax
import jax.numpy as jnp
from jax import lax
from jax.experimental import pallas as pl
from jax.experimental.pallas import tpu as pltpu
from jax.experimental.pallas import tpu_sc as plsc

BLK = 256
NC, NS = 2, 16
NW = NC * NS
CH = 32
MASK_HI = -65536


def _pack_rows(a):
    h = a.shape[1] // 2
    hi = lax.bitcast_convert_type(
        a[:, :h].astype(jnp.bfloat16).astype(jnp.float32), jnp.int32)
    lo = lax.bitcast_convert_type(
        a[:, h:].astype(jnp.bfloat16).astype(jnp.float32), jnp.int32)
    return hi | lax.shift_right_logical(lo, 16)


def _unpack_rows(p):
    hi = lax.bitcast_convert_type(p & MASK_HI, jnp.float32)
    lo = lax.bitcast_convert_type(lax.shift_left(p, 16), jnp.float32)
    return jnp.concatenate([hi, lo], axis=1)


def _router_meta_body(x_ref, wr_ref, pos_ref, w01_ref, be_ref, aux_ref,
                      xb_ref, xi_ref, *, n_tokens, n_experts, nb_tot):
    x = x_ref[...]
    xb_ref[...] = x.astype(jnp.bfloat16).reshape(xb_ref.shape)
    xi_ref[...] = _pack_rows(x)

    logits = lax.dot_general(
        x, wr_ref[...], (((1,), (1,)), ((), ())),
        preferred_element_type=jnp.float32)

    e_iota = lax.broadcasted_iota(jnp.int32, logits.shape, 1)
    m1 = jnp.max(logits, axis=-1, keepdims=True)
    i1 = jnp.min(jnp.where(logits == m1, e_iota, n_experts), axis=-1,
                 keepdims=True)
    oh1 = (e_iota == i1).astype(jnp.float32)
    masked = jnp.where(e_iota == i1, -jnp.inf, logits)
    m2 = jnp.max(masked, axis=-1, keepdims=True)
    i2 = jnp.min(jnp.where(masked == m2, e_iota, n_experts), axis=-1,
                 keepdims=True)
    oh2 = (e_iota == i2).astype(jnp.float32)
    w2 = 1.0 / (1.0 + jnp.exp(m1 - m2))
    w1 = 1.0 - w2
    w01_ref[...] = jnp.concatenate([w1, w2], axis=1)

    cnt = oh1 + oh2
    c = cnt
    s = 1
    while s < n_tokens:
        c = c + jnp.concatenate(
            [jnp.zeros((s, n_experts), jnp.float32), c[:-s, :]], axis=0)
        s *= 2
    c_excl = c - cnt
    counts = c[n_tokens - 1:n_tokens, :]
    ci = counts.astype(jnp.int32)
    ca = ((ci + (BLK - 1)) // BLK) * BLK
    off = ca
    s = 1
    while s < n_experts:
        off = off + jnp.concatenate(
            [jnp.zeros((1, s), jnp.int32), off[:, :-s]], axis=1)
        s *= 2
    off_excl_f = (off - ca).astype(jnp.float32)

    slot = off_excl_f + c_excl
    p0 = jnp.sum(oh1 * slot, axis=1, keepdims=True)
    p1 = jnp.sum(oh2 * slot, axis=1, keepdims=True)
    pos_ref[...] = jnp.concatenate([p0, p1], axis=1).astype(jnp.int32)

    bstart = lax.broadcasted_iota(jnp.int32, (nb_tot, n_experts), 0) * BLK
    be_ref[...] = jnp.sum(
        (jnp.broadcast_to(off, (nb_tot, n_experts)) <= bstart
         ).astype(jnp.int32), axis=1, keepdims=True)

    z = jnp.exp(logits - m1)
    sc = z / jnp.sum(z, axis=-1, keepdims=True)
    pi_sum = jnp.sum(sc, axis=0, keepdims=True)
    aux_ref[...] = (jnp.sum(pi_sum * counts)
                    / float(n_tokens * n_tokens)).reshape(1, 1)


def _gmm_body(be_ref, xp_ref, w_ref, y_ref, wb_ref, *, n_experts):
    i = pl.program_id(0)
    be = be_ref[i]
    prev = be_ref[jnp.maximum(i - 1, 0)]

    @pl.when((be < n_experts) & ((i == 0) | (be != prev)))
    def _cast():
        wb_ref[...] = w_ref[0].astype(jnp.bfloat16)

    @pl.when(be < n_experts)
    def _():
        xb = _unpack_rows(xp_ref[...]).astype(jnp.bfloat16)
        acc = lax.dot_general(xb, wb_ref[...], (((1,), (1,)), ((), ())),
                              preferred_element_type=jnp.float32)
        y_ref[...] = _pack_rows(acc)


def _sharedfinal_body(xb_ref, ws_ref, y0_ref, y1_ref, w01_ref, out_ref,
                      wsb_ref):
    @pl.when(pl.program_id(0) == 0)
    def _cast():
        wsb_ref[...] = (ws_ref[0] + ws_ref[1]).astype(jnp.bfloat16)

    blk = out_ref.shape[0]
    d = out_ref.shape[1]
    xb = xb_ref[...].reshape(blk, d)
    acc = lax.dot_general(xb, wsb_ref[...], (((1,), (1,)), ((), ())),
                          preferred_element_type=jnp.float32)
    w01 = w01_ref[...]
    y0 = _unpack_rows(y0_ref[...])
    y1 = _unpack_rows(y1_ref[...])
    out_ref[...] = acc + w01[:, 0:1] * y0 + w01[:, 1:2] * y1


def _make_dispatch(n_tokens, d, nk_pad):
    tpw = n_tokens // NW
    nch = tpw // CH
    h = d // 2
    mesh = plsc.VectorSubcoreMesh(core_axis_name="c", subcore_axis_name="s")

    @functools.partial(
        pl.kernel, mesh=mesh,
        out_type=jax.ShapeDtypeStruct((nk_pad, h), jnp.int32),
        scratch_types=[
            pltpu.VMEM((nch, 2, CH), jnp.int32),
            pltpu.VMEM((CH, h), jnp.int32),
            pltpu.VMEM((CH, h), jnp.int32),
        ] + [pltpu.SemaphoreType.DMA] * 6,
    )
    def dispatch(x_hbm, pos_hbm, xp_hbm, idx_v, buf0, buf1,
                 sr0, sr1, sw00, sw01, sw10, sw11):
        wid = lax.axis_index("s") * NC + lax.axis_index("c")
        base = wid * tpw
        pltpu.sync_copy(pos_hbm.at[wid], idx_v)
        bufs = (buf0, buf1)
        sr = (sr0, sr1)
        sw = ((sw00, sw01), (sw10, sw11))
        reads = [None] * nch
        writes = [None] * nch
        reads[0] = pltpu.async_copy(x_hbm.at[pl.ds(base, CH)], bufs[0], sr[0])
        for c in range(nch):
            b = c % 2
            reads[c].wait()
            if c + 1 < nch:
                if c >= 1:
                    writes[c - 1][0].wait()
                    writes[c - 1][1].wait()
                reads[c + 1] = pltpu.async_copy(
                    x_hbm.at[pl.ds(base + (c + 1) * CH, CH)],
                    bufs[1 - b], sr[1 - b])
            writes[c] = (
                pltpu.async_copy(bufs[b], xp_hbm.at[idx_v.at[c, 0]], sw[b][0]),
                pltpu.async_copy(bufs[b], xp_hbm.at[idx_v.at[c, 1]], sw[b][1]))
        writes[nch - 1][0].wait()
        writes[nch - 1][1].wait()
        if nch >= 2:
            writes[nch - 2][0].wait()
            writes[nch - 2][1].wait()

    return dispatch


def _make_combine(n_tokens, d, nk_pad):
    tpw = n_tokens // NW
    nch = tpw // CH
    h = d // 2
    mesh = plsc.VectorSubcoreMesh(core_axis_name="c", subcore_axis_name="s")

    @functools.partial(
        pl.kernel, mesh=mesh,
        out_type=(jax.ShapeDtypeStruct((n_tokens, h), jnp.int32),
                  jax.ShapeDtypeStruct((n_tokens, h), jnp.int32)),
        scratch_types=[
            pltpu.VMEM((nch, 2 * CH), jnp.int32),
            pltpu.VMEM((2 * CH, h), jnp.int32),
            pltpu.VMEM((2 * CH, h), jnp.int32),
        ] + [pltpu.SemaphoreType.DMA] * 6,
    )
    def combine(y_hbm, pos_hbm, y0_hbm, y1_hbm, idx_v, gbuf0, gbuf1,
                sg0, sg1, sw00, sw01, sw10, sw11):
        wid = lax.axis_index("s") * NC + lax.axis_index("c")
        base = wid * tpw
        pltpu.sync_copy(pos_hbm.at[wid], idx_v)
        gbufs = (gbuf0, gbuf1)
        sg = (sg0, sg1)
        sw = ((sw00, sw01), (sw10, sw11))
        reads = [None] * nch
        writes = [None] * nch
        reads[0] = pltpu.async_copy(y_hbm.at[idx_v.at[0]], gbufs[0], sg[0])
        for c in range(nch):
            b = c % 2
            reads[c].wait()
            if c + 1 < nch:
                if c >= 1:
                    writes[c - 1][0].wait()
                    writes[c - 1][1].wait()
                reads[c + 1] = pltpu.async_copy(
                    y_hbm.at[idx_v.at[c + 1]], gbufs[1 - b], sg[1 - b])
            dst = pl.ds(base + c * CH, CH)
            writes[c] = (
                pltpu.async_copy(gbufs[b].at[pl.ds(0, CH)],
                                 y0_hbm.at[dst], sw[b][0]),
                pltpu.async_copy(gbufs[b].at[pl.ds(CH, CH)],
                                 y1_hbm.at[dst], sw[b][1]))
        writes[nch - 1][0].wait()
        writes[nch - 1][1].wait()
        if nch >= 2:
            writes[nch - 2][0].wait()
            writes[nch - 2][1].wait()

    return combine


def _router_meta(x, W_router, nb_tot):
    n_tokens, d = x.shape
    n_experts = W_router.shape[0]
    return pl.pallas_call(
        functools.partial(_router_meta_body, n_tokens=n_tokens,
                          n_experts=n_experts, nb_tot=nb_tot),
        out_shape=[
            jax.ShapeDtypeStruct((n_tokens, 2), jnp.int32),
            jax.ShapeDtypeStruct((n_tokens, 2), jnp.float32),
            jax.ShapeDtypeStruct((nb_tot, 1), jnp.int32),
            jax.ShapeDtypeStruct((1, 1), jnp.float32),
            jax.ShapeDtypeStruct((n_tokens, d // 128, 128), jnp.bfloat16),
            jax.ShapeDtypeStruct((n_tokens, d // 2), jnp.int32),
        ],
    )(x, W_router)


def _gmm(be, x_perm, we, n_experts, d):
    nb_tot = be.shape[0]
    h = d // 2
    grid_spec = pltpu.PrefetchScalarGridSpec(
        num_scalar_prefetch=1,
        grid=(nb_tot,),
        in_specs=[
            pl.BlockSpec((BLK, h), lambda i, be_r: (i, 0)),
            pl.BlockSpec((1, d, d),
                         lambda i, be_r: (jnp.minimum(be_r[i], n_experts - 1),
                                          0, 0)),
        ],
        out_specs=pl.BlockSpec((BLK, h), lambda i, be_r: (i, 0)),
        scratch_shapes=[pltpu.VMEM((d, d), jnp.bfloat16)],
    )
    return pl.pallas_call(
        functools.partial(_gmm_body, n_experts=n_experts),
        grid_spec=grid_spec,
        out_shape=jax.ShapeDtypeStruct((x_perm.shape[0], h), jnp.int32),
    )(be, x_perm, we)


def _sharedfinal(xb, ws, y0, y1, w01):
    n_tokens = xb.shape[0]
    d = xb.shape[1] * xb.shape[2]
    dsub = d // 128
    h = d // 2
    blk = 512
    return pl.pallas_call(
        _sharedfinal_body,
        grid=(n_tokens // blk,),
        in_specs=[
            pl.BlockSpec((blk, dsub, 128), lambda i: (i, 0, 0)),
            pl.BlockSpec((2, d, d), lambda i: (0, 0, 0)),
            pl.BlockSpec((blk, h), lambda i: (i, 0)),
            pl.BlockSpec((blk, h), lambda i: (i, 0)),
            pl.BlockSpec((blk, 2), lambda i: (i, 0)),
        ],
        out_specs=pl.BlockSpec((blk, d), lambda i: (i, 0)),
        out_shape=jax.ShapeDtypeStruct((n_tokens, d), jnp.float32),
        scratch_shapes=[pltpu.VMEM((d, d), jnp.bfloat16)],
    )(xb, ws, y0, y1, w01)


def kernel(feat, W_router, W_shared, W_experts):
    B, S, d = feat.shape
    N = B * S
    E = W_router.shape[0]
    topk = 2
    nb_tot = (N * topk) // BLK + E
    nk_pad = nb_tot * BLK

    x = feat.reshape(N, d)
    ws = W_shared.reshape(-1, d, d)

    pos, w01, be2d, aux, xb, xi = _router_meta(x, W_router, nb_tot)
    tpw = N // NW
    nch = tpw // CH
    pos_sc = pos.reshape(NW, nch, CH, 2).transpose(0, 1, 3, 2)
    pos_disp = pos_sc
    pos_comb = pos_sc.reshape(NW, nch, 2 * CH)
    be = be2d.reshape(nb_tot)

    x_perm = _make_dispatch(N, d, nk_pad)(xi, pos_disp)
    y = _gmm(be, x_perm, W_experts, E, d)
    y0, y1 = _make_combine(N, d, nk_pad)(y, pos_comb)
    out = _sharedfinal(xb, ws, y0, y1, w01)
    return out.reshape(B, S, d), aux[0, 0]
